# baseline (speedup 1.0000x reference)
import functools

import jax
import jax.numpy as jnp
from jax import lax
from jax.experimental import pallas as pl
from jax.experimental.pallas import tpu as pltpu
from jax.experimental.pallas import tpu_sc as plsc

_NC = 2
_NS = 16
_NW = _NC * _NS
_L = 16


def _sc_body(C, RPW, x_hbm, t_hbm, xt_out, x0_out, rs_out,
             tgt_v, idx_t, idx_0, xt_v, x0_v, rs_v, rowbuf, sem):
    wid = lax.axis_index("s") * _NC + lax.axis_index("c")
    base = wid * RPW
    nchunk = RPW // _L
    ngather = RPW // 128

    pltpu.sync_copy(t_hbm.at[pl.ds(base, RPW)], tgt_v)

    lane = lax.iota(jnp.int32, _L)
    for j in range(nchunk):
        t = tgt_v[pl.ds(j * _L, _L)]
        g = (base + j * _L) + lane
        r, o = j // 8, (j % 8) * _L
        idx_t[r, pl.ds(o, _L)] = g * C + t
        idx_0[r, pl.ds(o, _L)] = g * C

    copies = []
    for j in range(ngather):
        copies.append(pltpu.async_copy(
            x_hbm.at[idx_t.at[j]], xt_v.at[pl.ds(j * 128, 128)], sem))
    for j in range(ngather):
        copies.append(pltpu.async_copy(
            x_hbm.at[idx_0.at[j]], x0_v.at[pl.ds(j * 128, 128)], sem))

    ones = jnp.ones((_L,), jnp.float32)
    nch = (C + _L - 1) // _L
    rem = C % _L
    if rem:
        rowbuf[pl.ds((nch - 1) * _L, _L)] = jnp.full((_L,), -1e30, jnp.float32)

    def scan_chunk(j, _):
        t = tgt_v[pl.ds(j * _L, _L)]
        v = ones
        for l in range(_L):
            def with_row(l=l, j=j):
                g = base + j * _L + l
                pltpu.sync_copy(x_hbm.at[pl.ds(g * C, C)],
                                rowbuf.at[pl.ds(0, C)])

                def acc_chunk(k, a):
                    return a + jnp.exp(rowbuf[pl.ds(k * _L, _L)])

                acc = lax.fori_loop(0, nch, acc_chunk,
                                    jnp.zeros((_L,), jnp.float32))
                s = acc[0]
                for q in range(1, _L):
                    s = s + acc[q]
                return s

            s = lax.cond(t[l] == 0, with_row, lambda: jnp.float32(1.0))
            v = jnp.where(lane == l, s, v)
        rs_v[pl.ds(j * _L, _L)] = v
        return 0

    lax.fori_loop(0, nchunk, scan_chunk, 0)

    for c in copies:
        c.wait()

    pltpu.sync_copy(xt_v, xt_out.at[pl.ds(base, RPW)])
    pltpu.sync_copy(x0_v, x0_out.at[pl.ds(base, RPW)])
    pltpu.sync_copy(rs_v, rs_out.at[pl.ds(base, RPW)])


def _combine_body(B, xt_ref, x0_ref, rs_ref, t_ref, out_ref):
    xt = xt_ref[...]
    x0 = x0_ref[...]
    d = x0 - xt
    sp = jnp.maximum(d, 0.0) + jnp.log(1.0 + jnp.exp(-jnp.abs(d)))
    lz = jnp.log(rs_ref[...]) - x0
    loss_rows = jnp.where(t_ref[...] == 0, lz, sp)
    out_ref[0, 0] = jnp.sum(loss_rows) / B


def kernel(inputs, targets):
    B, C = inputs.shape
    RPW = B // _NW
    t32 = targets.astype(jnp.int32)
    x_flat = inputs.reshape(B * C)

    nch = (C + _L - 1) // _L
    vec = jax.ShapeDtypeStruct((B,), jnp.float32)
    sc = pl.kernel(
        functools.partial(_sc_body, C, RPW),
        out_type=(vec, vec, vec),
        mesh=plsc.VectorSubcoreMesh(core_axis_name="c", subcore_axis_name="s"),
        scratch_types=[
            pltpu.VMEM((RPW,), jnp.int32),
            pltpu.VMEM((RPW // 128, 128), jnp.int32),
            pltpu.VMEM((RPW // 128, 128), jnp.int32),
            pltpu.VMEM((RPW,), jnp.float32),
            pltpu.VMEM((RPW,), jnp.float32),
            pltpu.VMEM((RPW,), jnp.float32),
            pltpu.VMEM((nch * _L,), jnp.float32),
            pltpu.SemaphoreType.DMA,
        ],
    )
    xt, x0, rs = sc(x_flat, t32)

    R = 128
    out = pl.pallas_call(
        functools.partial(_combine_body, B),
        out_specs=pl.BlockSpec(memory_space=pltpu.SMEM),
        out_shape=jax.ShapeDtypeStruct((1, 1), jnp.float32),
    )(xt.reshape(R, B // R), x0.reshape(R, B // R), rs.reshape(R, B // R),
      t32.reshape(R, B // R))
    return out[0, 0]


# speedup vs baseline: 1.3046x; 1.3046x over previous
import jax
import jax.numpy as jnp
from jax.experimental import pallas as pl
from jax.experimental.pallas import tpu as pltpu


def _body(x_ref, t_ref, out_ref):
    pid = pl.program_id(0)
    x = x_ref[...]
    t = t_ref[0, 0, :]
    blk, c = x.shape
    inv_b = 1.0 / (blk * pl.num_programs(0))

    col = jax.lax.broadcasted_iota(jnp.int32, (blk, c), 1)
    xt = jnp.sum(jnp.where(col == t[:, None], x, 0.0), axis=1)
    x0 = x[:, 0]

    d = x0 - xt
    sp = jnp.maximum(d, 0.0) + jnp.log(1.0 + jnp.exp(-jnp.abs(d)))

    @pl.when(pid == 0)
    def _():
        out_ref[0, 0] = 0.0

    out_ref[0, 0] += jnp.sum(jnp.where(t == 0, 0.0, sp)) * inv_b

    @pl.when(jnp.min(t) == 0)
    def _():
        s = jnp.sum(jnp.exp(x), axis=1)
        lz = jnp.log(s) - x0
        out_ref[0, 0] += jnp.sum(jnp.where(t == 0, lz, 0.0)) * inv_b


def kernel(inputs, targets):
    B, C = inputs.shape
    BLK = 256
    grid = B // BLK
    t3 = targets.astype(jnp.int32).reshape(grid, 1, BLK)

    out = pl.pallas_call(
        _body,
        grid=(grid,),
        in_specs=[
            pl.BlockSpec((BLK, C), lambda i: (i, 0)),
            pl.BlockSpec((1, 1, BLK), lambda i: (i, 0, 0)),
        ],
        out_specs=pl.BlockSpec(memory_space=pltpu.SMEM),
        out_shape=jax.ShapeDtypeStruct((1, 1), jnp.float32),
    )(inputs, t3)
    return out[0, 0]
